# baseline (speedup 1.0000x reference)
import functools

import jax
import jax.numpy as jnp
from jax import lax
from jax.experimental import pallas as pl
from jax.experimental.pallas import tpu as pltpu
from jax.experimental.pallas import tpu_sc as plsc

_SIGMA_DIM = 4
_EMB_DIM = 32
_ROW = _SIGMA_DIM + _EMB_DIM
_L = 16
_CH = 128
_SL = 5


def _body(nc, bpw, keys_hbm, table_hbm, params_hbm, out_hbm,
          keys_v, idx_v, sl_v, out_v, params_v, sem):
  wid = lax.axis_index("s") * nc + lax.axis_index("c")
  base = wid * bpw
  ngr = bpw // _L

  pltpu.sync_copy(keys_hbm.at[pl.ds(base, bpw)], keys_v)
  pltpu.sync_copy(params_hbm, params_v)

  def build(g, carry):
    i16 = g * _L + lax.iota(jnp.int32, _L)
    k16 = keys_v[pl.ds(g * _L, _L)]
    s0 = (k16 * _ROW) >> 3
    p = i16 * _SL
    for j in range(_SL):
      pj = p + j
      plsc.store_scatter(idx_v, [pj >> 7, pj & (_CH - 1)], s0 + j)
    return carry

  lax.fori_loop(0, ngr, build, 0)

  nch = (bpw * _SL) // _CH
  copies = [
      pltpu.async_copy(table_hbm.at[idx_v.at[j]],
                       sl_v.at[pl.ds(j * _CH, _CH)], sem)
      for j in range(nch)
  ]
  for c in copies:
    c.wait()

  p0 = params_v[pl.ds(0, _L)]
  p1 = params_v[pl.ds(_L, _L)]
  p2 = params_v[pl.ds(2 * _L, _L)]
  vc = [p0[c] for c in range(_L)] + [p1[c] for c in range(_L)]
  sk = [p2[c] for c in range(_SIGMA_DIM)]
  bias = p2[_SIGMA_DIM]

  def group(g, carry):
    i16 = g * _L + lax.iota(jnp.int32, _L)
    k16 = keys_v[pl.ds(g * _L, _L)]
    wbase = i16 * (_SL * 8) + ((k16 & 1) << 2)
    acc = jnp.full((_L,), 0.0, jnp.float32)
    for c in range(_SIGMA_DIM):
      w = wbase + c
      acc = acc + sk[c] * plsc.load_gather(sl_v, [w >> 3, w & 7])
    s = 1.0 / (1.0 + jnp.exp(-(acc + bias)))
    one_m_s = 1.0 - s
    for c in range(_EMB_DIM):
      w = wbase + (_SIGMA_DIM + c)
      v = plsc.load_gather(sl_v, [w >> 3, w & 7])
      o = s * vc[c] + one_m_s * v
      plsc.store_scatter(out_v, [i16, jnp.full((_L,), c, jnp.int32)], o)
    return carry

  lax.fori_loop(0, ngr, group, 0)

  pltpu.sync_copy(out_v, out_hbm.at[pl.ds(base, bpw)])


def kernel(keys, table, vc, sigma_kernel, sigma_bias):
  keys = keys.astype(jnp.int32)
  num_emb = table.shape[0]
  table8 = table.reshape(num_emb * _ROW // 8, 8)
  params = jnp.concatenate(
      [vc, sigma_kernel, sigma_bias, jnp.zeros((11,), jnp.float32)])
  info = plsc.get_sparse_core_info()
  nc, ns = info.num_cores, info.num_subcores
  nw = nc * ns
  batch = keys.shape[0]
  bpw = batch // nw

  mesh = plsc.VectorSubcoreMesh(core_axis_name="c", subcore_axis_name="s")
  run = pl.kernel(
      functools.partial(_body, nc, bpw),
      out_type=jax.ShapeDtypeStruct((batch, _EMB_DIM), jnp.float32),
      mesh=mesh,
      compiler_params=pltpu.CompilerParams(
          needs_layout_passes=False, use_tc_tiling_on_sc=False),
      scratch_types=[
          pltpu.VMEM((bpw,), jnp.int32),
          pltpu.VMEM((bpw * _SL // _CH, _CH), jnp.int32),
          pltpu.VMEM((bpw * _SL, 8), jnp.float32),
          pltpu.VMEM((bpw, _EMB_DIM), jnp.float32),
          pltpu.VMEM((48,), jnp.float32),
          pltpu.SemaphoreType.DMA,
      ],
  )
  return run(keys, table8, params)
